# Initial kernel scaffold; baseline (speedup 1.0000x reference)
import jax
import jax.numpy as jnp
from jax.experimental import pallas as pl


def kernel(x, edge_index, W_l, b_l, W_r, gn_weight, gn_bias, gn_mean_scale):
    raise NotImplementedError("write your pallas kernel here")


# speedup vs baseline: 8.1977x; 8.1977x over previous
import functools

import jax
import jax.numpy as jnp
from jax import lax
from jax.experimental import pallas as pl
from jax.experimental.pallas import tpu as pltpu
from jax.experimental.pallas import tpu_sc as plsc

N = 10000
D = 128
E = 320000
NC = 2
NS = 16
NW = NC * NS
EPW = E // NW
CHUNK = 80
NCHUNK = EPW // CHUNK
RPW = 624
NP = 10240


def _sc_body(x_hbm, src_hbm, dst_hbm, agg_out, cnt_out,
             src_idx, dst_idx, rows, ones, zcnt,
             agg_sh, cnt_sh, sem):
    cid = lax.axis_index("c")
    sid = lax.axis_index("s")
    wid = cid * NS + sid

    def zr_body(r, carry):
        for c in range(D // 16):
            rows[r, pl.ds(c * 16, 16)] = jnp.zeros((16,), jnp.float32)
        return carry
    lax.fori_loop(0, CHUNK, zr_body, 0)

    def zc_body(i, carry):
        zcnt[pl.ds(i * 16, 16)] = jnp.zeros((16,), jnp.float32)
        return carry
    lax.fori_loop(0, 1024 // 16, zc_body, 0)

    for c in range(CHUNK // 16):
        ones[pl.ds(c * 16, 16)] = jnp.ones((16,), jnp.float32)

    row0 = sid * RPW
    for k in range(RPW // CHUNK):
        pltpu.sync_copy(rows, agg_sh.at[pl.ds(row0 + k * CHUNK, CHUNK)])
    pltpu.sync_copy(rows.at[pl.ds(0, RPW % CHUNK)],
                    agg_sh.at[pl.ds(row0 + RPW - RPW % CHUNK, RPW % CHUNK)])

    @pl.when(sid == NS - 1)
    def _():
        pltpu.sync_copy(rows.at[pl.ds(0, 16)],
                        agg_sh.at[pl.ds(NS * RPW, 16)])

    @pl.when(sid == 0)
    def _():
        for k in range(NP // 1024):
            pltpu.sync_copy(zcnt, cnt_sh.at[pl.ds(k * 1024, 1024)])

    plsc.subcore_barrier()

    pltpu.sync_copy(src_hbm.at[wid], src_idx)
    pltpu.sync_copy(dst_hbm.at[wid], dst_idx)

    def chunk_body(j, carry):
        pltpu.async_copy(x_hbm.at[src_idx.at[j]], rows, sem).wait()
        pltpu.sync_copy(rows, agg_sh.at[dst_idx.at[j]], add=True)
        pltpu.sync_copy(ones, cnt_sh.at[dst_idx.at[j]], add=True)
        return carry
    lax.fori_loop(0, NCHUNK, chunk_body, 0)

    plsc.subcore_barrier()

    pltpu.sync_copy(agg_sh.at[pl.ds(row0, RPW)],
                    agg_out.at[cid, pl.ds(row0, RPW)])

    @pl.when(sid == NS - 1)
    def _():
        pltpu.sync_copy(agg_sh.at[pl.ds(NS * RPW, 16)],
                        agg_out.at[cid, pl.ds(NS * RPW, 16)])

    @pl.when(sid == 0)
    def _():
        pltpu.sync_copy(cnt_sh, cnt_out.at[pl.ds(cid * NP, NP)])


_sc_segment_sum = functools.partial(
    pl.kernel,
    out_type=(jax.ShapeDtypeStruct((NC, N, D), jnp.float32),
              jax.ShapeDtypeStruct((NC * NP,), jnp.float32)),
    mesh=plsc.VectorSubcoreMesh(core_axis_name="c", subcore_axis_name="s"),
    scratch_types=[
        pltpu.VMEM((NCHUNK, CHUNK), jnp.int32),
        pltpu.VMEM((NCHUNK, CHUNK), jnp.int32),
        pltpu.VMEM((CHUNK, D), jnp.float32),
        pltpu.VMEM((CHUNK,), jnp.float32),
        pltpu.VMEM((1024,), jnp.float32),
        pltpu.VMEM_SHARED((N, D), jnp.float32),
        pltpu.VMEM_SHARED((NP,), jnp.float32),
        pltpu.SemaphoreType.DMA,
    ],
)(_sc_body)


R = 1000
NB = N // R


def _dense_body(agg_ref, cnt_ref, x_ref, wl_ref, wr_ref, bl_ref,
                h_ref, stats_ref):
    i = pl.program_id(0)
    agg = agg_ref[0] + agg_ref[1]
    c = cnt_ref[0] + cnt_ref[1]
    mean = agg * (1.0 / jnp.maximum(c, 1.0))
    dn = (((1,), (1,)), ((), ()))
    h = (lax.dot_general(mean, wl_ref[...], dn,
                         preferred_element_type=jnp.float32)
         + lax.dot_general(x_ref[...], wr_ref[...], dn,
                           preferred_element_type=jnp.float32)
         + bl_ref[...])
    h_ref[...] = h
    sh = jnp.sum(h, axis=0)[None]
    sh2 = jnp.sum(h * h, axis=0)[None]
    upd = jnp.concatenate([sh, sh2, jnp.zeros((6, D), jnp.float32)], axis=0)
    prev = jnp.where(i == 0, jnp.zeros_like(upd), stats_ref[...])
    stats_ref[...] = prev + upd


def _tc_dense(agg_parts, cnt_parts, x, W_l, W_r, b_l):
    return pl.pallas_call(
        _dense_body,
        grid=(NB,),
        in_specs=[
            pl.BlockSpec((NC, R, D), lambda i: (0, i, 0)),
            pl.BlockSpec((NC, R, 1), lambda i: (0, i, 0)),
            pl.BlockSpec((R, D), lambda i: (i, 0)),
            pl.BlockSpec((D, D), lambda i: (0, 0)),
            pl.BlockSpec((D, D), lambda i: (0, 0)),
            pl.BlockSpec((1, D), lambda i: (0, 0)),
        ],
        out_specs=[
            pl.BlockSpec((R, D), lambda i: (i, 0)),
            pl.BlockSpec((8, D), lambda i: (0, 0)),
        ],
        out_shape=[
            jax.ShapeDtypeStruct((N, D), jnp.float32),
            jax.ShapeDtypeStruct((8, D), jnp.float32),
        ],
    )(agg_parts, cnt_parts, x, W_l, W_r, b_l)


def _norm_body(h_ref, stats_ref, x_ref, w_ref, b_ref, ms_ref, o_ref):
    h = h_ref[...]
    stats = stats_ref[...]
    mu = stats[0:1] * (1.0 / N)
    m2 = stats[1:2] * (1.0 / N)
    mus = mu * ms_ref[...]
    var = m2 - 2.0 * mus * mu + mus * mus
    rstd = lax.rsqrt(var + 1e-5)
    hn = (h - mus) * rstd * w_ref[...] + b_ref[...]
    g = 0.5 * hn * (1.0 + lax.erf(hn * 0.7071067811865476))
    o_ref[...] = g + x_ref[...]


def _tc_norm(h, stats, x, gn_weight, gn_bias, gn_mean_scale):
    return pl.pallas_call(
        _norm_body,
        grid=(NB,),
        in_specs=[
            pl.BlockSpec((R, D), lambda i: (i, 0)),
            pl.BlockSpec((8, D), lambda i: (0, 0)),
            pl.BlockSpec((R, D), lambda i: (i, 0)),
            pl.BlockSpec((1, D), lambda i: (0, 0)),
            pl.BlockSpec((1, D), lambda i: (0, 0)),
            pl.BlockSpec((1, D), lambda i: (0, 0)),
        ],
        out_specs=pl.BlockSpec((R, D), lambda i: (i, 0)),
        out_shape=jax.ShapeDtypeStruct((N, D), jnp.float32),
    )(h, stats, x, gn_weight, gn_bias, gn_mean_scale)


def kernel(x, edge_index, W_l, b_l, W_r, gn_weight, gn_bias, gn_mean_scale):
    src = edge_index[0].reshape(NW, NCHUNK, CHUNK)
    dst = edge_index[1].reshape(NW, NCHUNK, CHUNK)
    agg_parts, cnt_flat = _sc_segment_sum(x, src, dst)
    cnt_parts = cnt_flat.reshape(NC, NP)[:, :N].reshape(NC, N, 1)
    h, stats = _tc_dense(agg_parts, cnt_parts, x, W_l, W_r,
                         b_l.reshape(1, D))
    return _tc_norm(h, stats, x, gn_weight.reshape(1, D),
                    gn_bias.reshape(1, D), gn_mean_scale.reshape(1, D))


# speedup vs baseline: 10.2133x; 1.2459x over previous
import functools

import jax
import jax.numpy as jnp
from jax import lax
from jax.experimental import pallas as pl
from jax.experimental.pallas import tpu as pltpu
from jax.experimental.pallas import tpu_sc as plsc

N = 10000
D = 128
E = 320000
NC = 2
NS = 16
NW = NC * NS
EPW = E // NW
CHUNK = 50
NCHUNK = EPW // CHUNK
BLK = 40
NBLK = NCHUNK // BLK
RPW = 624
NP = 10240


def _sc_body(x_hbm, src_hbm, dst_hbm, agg_out, cnt_out,
             srcA, dstA, srcB, dstB, rows, rows2, ones, zcnt,
             agg_sh, cnt_sh, siA, diA, siB, diB, sem, sem2):
    cid = lax.axis_index("c")
    sid = lax.axis_index("s")
    wid = cid * NS + sid

    def zr_body(r, carry):
        for c in range(D // 16):
            rows[r, pl.ds(c * 16, 16)] = jnp.zeros((16,), jnp.float32)
        return carry
    lax.fori_loop(0, CHUNK, zr_body, 0)

    def zc_body(i, carry):
        zcnt[pl.ds(i * 16, 16)] = jnp.zeros((16,), jnp.float32)
        return carry
    lax.fori_loop(0, 1024 // 16, zc_body, 0)

    for c in range(64 // 16):
        ones[pl.ds(c * 16, 16)] = jnp.ones((16,), jnp.float32)

    row0 = sid * RPW
    for k in range(RPW // CHUNK):
        pltpu.sync_copy(rows, agg_sh.at[pl.ds(row0 + k * CHUNK, CHUNK)])
    pltpu.sync_copy(rows.at[pl.ds(0, RPW % CHUNK)],
                    agg_sh.at[pl.ds(row0 + RPW - RPW % CHUNK, RPW % CHUNK)])

    @pl.when(sid == NS - 1)
    def _():
        pltpu.sync_copy(rows.at[pl.ds(0, 16)],
                        agg_sh.at[pl.ds(NS * RPW, 16)])

    @pl.when(sid == 0)
    def _():
        for k in range(NP // 1024):
            pltpu.sync_copy(zcnt, cnt_sh.at[pl.ds(k * 1024, 1024)])

    plsc.subcore_barrier()

    bufs = [(srcA, dstA, siA, diA), (srcB, dstB, siB, diB)]

    def stage(b, bi):
        sb, db, ss, ds_ = bufs[bi]
        return (pltpu.make_async_copy(src_hbm.at[wid, pl.ds(b * BLK, BLK)],
                                      sb, ss),
                pltpu.make_async_copy(dst_hbm.at[wid, pl.ds(b * BLK, BLK)],
                                      db, ds_))

    for c in stage(0, 0):
        c.start()
    for b in range(NBLK):
        bi = b % 2
        if b + 1 < NBLK:
            for c in stage(b + 1, (b + 1) % 2):
                c.start()
        for c in stage(b, bi):
            c.wait()
        sb, db = bufs[bi][0], bufs[bi][1]

        def gather(j, buf, s):
            return pltpu.make_async_copy(x_hbm.at[sb.at[j]], buf, s)

        gather(0, rows, sem).start()

        def pair_body(i, carry):
            j = i * 2
            gather(j + 1, rows2, sem2).start()
            gather(j, rows, sem).wait()
            pltpu.sync_copy(rows, agg_sh.at[db.at[j]], add=True)
            pltpu.sync_copy(ones.at[pl.ds(0, CHUNK)], cnt_sh.at[db.at[j]],
                            add=True)
            jn = jnp.minimum(j + 2, BLK - 1)
            gather(jn, rows, sem).start()
            gather(j + 1, rows2, sem2).wait()
            pltpu.sync_copy(rows2, agg_sh.at[db.at[j + 1]], add=True)
            pltpu.sync_copy(ones.at[pl.ds(0, CHUNK)],
                            cnt_sh.at[db.at[j + 1]], add=True)
            return carry
        lax.fori_loop(0, BLK // 2, pair_body, 0)
        gather(BLK - 1, rows, sem).wait()

    plsc.subcore_barrier()

    pltpu.sync_copy(agg_sh.at[pl.ds(row0, RPW)],
                    agg_out.at[cid, pl.ds(row0, RPW)])

    @pl.when(sid == NS - 1)
    def _():
        pltpu.sync_copy(agg_sh.at[pl.ds(NS * RPW, 16)],
                        agg_out.at[cid, pl.ds(NS * RPW, 16)])

    @pl.when(sid == 0)
    def _():
        pltpu.sync_copy(cnt_sh, cnt_out.at[pl.ds(cid * NP, NP)])


_sc_segment_sum = functools.partial(
    pl.kernel,
    out_type=(jax.ShapeDtypeStruct((NC, N, D), jnp.float32),
              jax.ShapeDtypeStruct((NC * NP,), jnp.float32)),
    mesh=plsc.VectorSubcoreMesh(core_axis_name="c", subcore_axis_name="s"),
    scratch_types=[
        pltpu.VMEM((BLK, CHUNK), jnp.int32),
        pltpu.VMEM((BLK, CHUNK), jnp.int32),
        pltpu.VMEM((BLK, CHUNK), jnp.int32),
        pltpu.VMEM((BLK, CHUNK), jnp.int32),
        pltpu.VMEM((CHUNK, D), jnp.float32),
        pltpu.VMEM((CHUNK, D), jnp.float32),
        pltpu.VMEM((64,), jnp.float32),
        pltpu.VMEM((1024,), jnp.float32),
        pltpu.VMEM_SHARED((N, D), jnp.float32),
        pltpu.VMEM_SHARED((NP,), jnp.float32),
        pltpu.SemaphoreType.DMA,
        pltpu.SemaphoreType.DMA,
        pltpu.SemaphoreType.DMA,
        pltpu.SemaphoreType.DMA,
        pltpu.SemaphoreType.DMA,
        pltpu.SemaphoreType.DMA,
    ],
)(_sc_body)


R = 1000
NB = N // R


def _dense_body(agg_ref, cnt_ref, x_ref, wl_ref, wr_ref, bl_ref,
                h_ref, stats_ref):
    i = pl.program_id(0)
    agg = agg_ref[0] + agg_ref[1]
    c = cnt_ref[0] + cnt_ref[1]
    mean = agg * (1.0 / jnp.maximum(c, 1.0))
    dn = (((1,), (1,)), ((), ()))
    h = (lax.dot_general(mean, wl_ref[...], dn,
                         preferred_element_type=jnp.float32)
         + lax.dot_general(x_ref[...], wr_ref[...], dn,
                           preferred_element_type=jnp.float32)
         + bl_ref[...])
    h_ref[...] = h
    sh = jnp.sum(h, axis=0)[None]
    sh2 = jnp.sum(h * h, axis=0)[None]
    upd = jnp.concatenate([sh, sh2, jnp.zeros((6, D), jnp.float32)], axis=0)
    prev = jnp.where(i == 0, jnp.zeros_like(upd), stats_ref[...])
    stats_ref[...] = prev + upd


def _tc_dense(agg_parts, cnt_parts, x, W_l, W_r, b_l):
    return pl.pallas_call(
        _dense_body,
        grid=(NB,),
        in_specs=[
            pl.BlockSpec((NC, R, D), lambda i: (0, i, 0)),
            pl.BlockSpec((NC, R, 1), lambda i: (0, i, 0)),
            pl.BlockSpec((R, D), lambda i: (i, 0)),
            pl.BlockSpec((D, D), lambda i: (0, 0)),
            pl.BlockSpec((D, D), lambda i: (0, 0)),
            pl.BlockSpec((1, D), lambda i: (0, 0)),
        ],
        out_specs=[
            pl.BlockSpec((R, D), lambda i: (i, 0)),
            pl.BlockSpec((8, D), lambda i: (0, 0)),
        ],
        out_shape=[
            jax.ShapeDtypeStruct((N, D), jnp.float32),
            jax.ShapeDtypeStruct((8, D), jnp.float32),
        ],
    )(agg_parts, cnt_parts, x, W_l, W_r, b_l)


def _norm_body(h_ref, stats_ref, x_ref, w_ref, b_ref, ms_ref, o_ref):
    h = h_ref[...]
    stats = stats_ref[...]
    mu = stats[0:1] * (1.0 / N)
    m2 = stats[1:2] * (1.0 / N)
    mus = mu * ms_ref[...]
    var = m2 - 2.0 * mus * mu + mus * mus
    rstd = lax.rsqrt(var + 1e-5)
    hn = (h - mus) * rstd * w_ref[...] + b_ref[...]
    g = 0.5 * hn * (1.0 + lax.erf(hn * 0.7071067811865476))
    o_ref[...] = g + x_ref[...]


def _tc_norm(h, stats, x, gn_weight, gn_bias, gn_mean_scale):
    return pl.pallas_call(
        _norm_body,
        grid=(NB,),
        in_specs=[
            pl.BlockSpec((R, D), lambda i: (i, 0)),
            pl.BlockSpec((8, D), lambda i: (0, 0)),
            pl.BlockSpec((R, D), lambda i: (i, 0)),
            pl.BlockSpec((1, D), lambda i: (0, 0)),
            pl.BlockSpec((1, D), lambda i: (0, 0)),
            pl.BlockSpec((1, D), lambda i: (0, 0)),
        ],
        out_specs=pl.BlockSpec((R, D), lambda i: (i, 0)),
        out_shape=jax.ShapeDtypeStruct((N, D), jnp.float32),
    )(h, stats, x, gn_weight, gn_bias, gn_mean_scale)


def kernel(x, edge_index, W_l, b_l, W_r, gn_weight, gn_bias, gn_mean_scale):
    src = edge_index[0].reshape(NW, NCHUNK, CHUNK)
    dst = edge_index[1].reshape(NW, NCHUNK, CHUNK)
    agg_parts, cnt_flat = _sc_segment_sum(x, src, dst)
    cnt_parts = cnt_flat.reshape(NC, NP)[:, :N].reshape(NC, N, 1)
    h, stats = _tc_dense(agg_parts, cnt_parts, x, W_l, W_r,
                         b_l.reshape(1, D))
    return _tc_norm(h, stats, x, gn_weight.reshape(1, D),
                    gn_bias.reshape(1, D), gn_mean_scale.reshape(1, D))


# speedup vs baseline: 12.5014x; 1.2240x over previous
import functools

import jax
import jax.numpy as jnp
from jax import lax
from jax.experimental import pallas as pl
from jax.experimental.pallas import tpu as pltpu
from jax.experimental.pallas import tpu_sc as plsc

N = 10000
D = 128
E = 320000
NC = 2
NS = 16
NW = NC * NS
EPW = E // NW
CHUNK = 125
NCHUNK = EPW // CHUNK
BLK = 16
NBLK = NCHUNK // BLK
RPW = 624
NP = 10240


def _sc_body(x_hbm, src_hbm, dst_hbm, agg_out, cnt_out,
             srcA, dstA, srcB, dstB, rows, rows2, ones, zcnt,
             agg_sh, cnt_sh, siA, diA, siB, diB, sem, sem2):
    cid = lax.axis_index("c")
    sid = lax.axis_index("s")
    wid = cid * NS + sid

    def zr_body(r, carry):
        for c in range(D // 16):
            rows[r, pl.ds(c * 16, 16)] = jnp.zeros((16,), jnp.float32)
        return carry
    lax.fori_loop(0, CHUNK, zr_body, 0)

    def zc_body(i, carry):
        zcnt[pl.ds(i * 16, 16)] = jnp.zeros((16,), jnp.float32)
        return carry
    lax.fori_loop(0, 1024 // 16, zc_body, 0)

    for c in range(128 // 16):
        ones[pl.ds(c * 16, 16)] = jnp.ones((16,), jnp.float32)

    row0 = sid * RPW
    for k in range(RPW // CHUNK):
        pltpu.sync_copy(rows, agg_sh.at[pl.ds(row0 + k * CHUNK, CHUNK)])
    pltpu.sync_copy(rows.at[pl.ds(0, RPW % CHUNK)],
                    agg_sh.at[pl.ds(row0 + RPW - RPW % CHUNK, RPW % CHUNK)])

    @pl.when(sid == NS - 1)
    def _():
        pltpu.sync_copy(rows.at[pl.ds(0, 16)],
                        agg_sh.at[pl.ds(NS * RPW, 16)])

    @pl.when(sid == 0)
    def _():
        for k in range(NP // 1024):
            pltpu.sync_copy(zcnt, cnt_sh.at[pl.ds(k * 1024, 1024)])

    plsc.subcore_barrier()

    bufs = [(srcA, dstA, siA, diA), (srcB, dstB, siB, diB)]

    def stage(b, bi):
        sb, db, ss, ds_ = bufs[bi]
        return (pltpu.make_async_copy(src_hbm.at[wid, pl.ds(b * BLK, BLK)],
                                      sb, ss),
                pltpu.make_async_copy(dst_hbm.at[wid, pl.ds(b * BLK, BLK)],
                                      db, ds_))

    for c in stage(0, 0):
        c.start()
    for b in range(NBLK):
        bi = b % 2
        if b + 1 < NBLK:
            for c in stage(b + 1, (b + 1) % 2):
                c.start()
        for c in stage(b, bi):
            c.wait()
        sb, db = bufs[bi][0], bufs[bi][1]

        def gather(j, buf, s):
            return pltpu.make_async_copy(x_hbm.at[sb.at[j]], buf, s)

        gather(0, rows, sem).start()

        def pair_body(i, carry):
            j = i * 2
            gather(j + 1, rows2, sem2).start()
            gather(j, rows, sem).wait()
            pltpu.sync_copy(rows, agg_sh.at[db.at[j]], add=True)
            pltpu.sync_copy(ones.at[pl.ds(0, CHUNK)], cnt_sh.at[db.at[j]],
                            add=True)
            jn = jnp.minimum(j + 2, BLK - 1)
            gather(jn, rows, sem).start()
            gather(j + 1, rows2, sem2).wait()
            pltpu.sync_copy(rows2, agg_sh.at[db.at[j + 1]], add=True)
            pltpu.sync_copy(ones.at[pl.ds(0, CHUNK)],
                            cnt_sh.at[db.at[j + 1]], add=True)
            return carry
        lax.fori_loop(0, BLK // 2, pair_body, 0)
        gather(BLK - 1, rows, sem).wait()

    plsc.subcore_barrier()

    pltpu.sync_copy(agg_sh.at[pl.ds(row0, RPW)],
                    agg_out.at[cid, pl.ds(row0, RPW)])

    @pl.when(sid == NS - 1)
    def _():
        pltpu.sync_copy(agg_sh.at[pl.ds(NS * RPW, 16)],
                        agg_out.at[cid, pl.ds(NS * RPW, 16)])

    @pl.when(sid == 0)
    def _():
        pltpu.sync_copy(cnt_sh, cnt_out.at[pl.ds(cid * NP, NP)])


_sc_segment_sum = functools.partial(
    pl.kernel,
    out_type=(jax.ShapeDtypeStruct((NC, N, D), jnp.float32),
              jax.ShapeDtypeStruct((NC * NP,), jnp.float32)),
    mesh=plsc.VectorSubcoreMesh(core_axis_name="c", subcore_axis_name="s"),
    scratch_types=[
        pltpu.VMEM((BLK, CHUNK), jnp.int32),
        pltpu.VMEM((BLK, CHUNK), jnp.int32),
        pltpu.VMEM((BLK, CHUNK), jnp.int32),
        pltpu.VMEM((BLK, CHUNK), jnp.int32),
        pltpu.VMEM((CHUNK, D), jnp.float32),
        pltpu.VMEM((CHUNK, D), jnp.float32),
        pltpu.VMEM((128,), jnp.float32),
        pltpu.VMEM((1024,), jnp.float32),
        pltpu.VMEM_SHARED((N, D), jnp.float32),
        pltpu.VMEM_SHARED((NP,), jnp.float32),
        pltpu.SemaphoreType.DMA,
        pltpu.SemaphoreType.DMA,
        pltpu.SemaphoreType.DMA,
        pltpu.SemaphoreType.DMA,
        pltpu.SemaphoreType.DMA,
        pltpu.SemaphoreType.DMA,
    ],
)(_sc_body)


R = 1000
NB = N // R


def _dense_body(agg_ref, cnt_ref, x_ref, wl_ref, wr_ref, bl_ref,
                h_ref, stats_ref):
    i = pl.program_id(0)
    agg = agg_ref[0] + agg_ref[1]
    c = cnt_ref[0] + cnt_ref[1]
    mean = agg * (1.0 / jnp.maximum(c, 1.0))
    dn = (((1,), (1,)), ((), ()))
    h = (lax.dot_general(mean, wl_ref[...], dn,
                         preferred_element_type=jnp.float32)
         + lax.dot_general(x_ref[...], wr_ref[...], dn,
                           preferred_element_type=jnp.float32)
         + bl_ref[...])
    h_ref[...] = h
    sh = jnp.sum(h, axis=0)[None]
    sh2 = jnp.sum(h * h, axis=0)[None]
    upd = jnp.concatenate([sh, sh2, jnp.zeros((6, D), jnp.float32)], axis=0)
    prev = jnp.where(i == 0, jnp.zeros_like(upd), stats_ref[...])
    stats_ref[...] = prev + upd


def _tc_dense(agg_parts, cnt_parts, x, W_l, W_r, b_l):
    return pl.pallas_call(
        _dense_body,
        grid=(NB,),
        in_specs=[
            pl.BlockSpec((NC, R, D), lambda i: (0, i, 0)),
            pl.BlockSpec((NC, R, 1), lambda i: (0, i, 0)),
            pl.BlockSpec((R, D), lambda i: (i, 0)),
            pl.BlockSpec((D, D), lambda i: (0, 0)),
            pl.BlockSpec((D, D), lambda i: (0, 0)),
            pl.BlockSpec((1, D), lambda i: (0, 0)),
        ],
        out_specs=[
            pl.BlockSpec((R, D), lambda i: (i, 0)),
            pl.BlockSpec((8, D), lambda i: (0, 0)),
        ],
        out_shape=[
            jax.ShapeDtypeStruct((N, D), jnp.float32),
            jax.ShapeDtypeStruct((8, D), jnp.float32),
        ],
    )(agg_parts, cnt_parts, x, W_l, W_r, b_l)


def _norm_body(h_ref, stats_ref, x_ref, w_ref, b_ref, ms_ref, o_ref):
    h = h_ref[...]
    stats = stats_ref[...]
    mu = stats[0:1] * (1.0 / N)
    m2 = stats[1:2] * (1.0 / N)
    mus = mu * ms_ref[...]
    var = m2 - 2.0 * mus * mu + mus * mus
    rstd = lax.rsqrt(var + 1e-5)
    hn = (h - mus) * rstd * w_ref[...] + b_ref[...]
    g = 0.5 * hn * (1.0 + lax.erf(hn * 0.7071067811865476))
    o_ref[...] = g + x_ref[...]


def _tc_norm(h, stats, x, gn_weight, gn_bias, gn_mean_scale):
    return pl.pallas_call(
        _norm_body,
        grid=(NB,),
        in_specs=[
            pl.BlockSpec((R, D), lambda i: (i, 0)),
            pl.BlockSpec((8, D), lambda i: (0, 0)),
            pl.BlockSpec((R, D), lambda i: (i, 0)),
            pl.BlockSpec((1, D), lambda i: (0, 0)),
            pl.BlockSpec((1, D), lambda i: (0, 0)),
            pl.BlockSpec((1, D), lambda i: (0, 0)),
        ],
        out_specs=pl.BlockSpec((R, D), lambda i: (i, 0)),
        out_shape=jax.ShapeDtypeStruct((N, D), jnp.float32),
    )(h, stats, x, gn_weight, gn_bias, gn_mean_scale)


def kernel(x, edge_index, W_l, b_l, W_r, gn_weight, gn_bias, gn_mean_scale):
    src = edge_index[0].reshape(NW, NCHUNK, CHUNK)
    dst = edge_index[1].reshape(NW, NCHUNK, CHUNK)
    agg_parts, cnt_flat = _sc_segment_sum(x, src, dst)
    cnt_parts = cnt_flat.reshape(NC, NP)[:, :N].reshape(NC, N, 1)
    h, stats = _tc_dense(agg_parts, cnt_parts, x, W_l, W_r,
                         b_l.reshape(1, D))
    return _tc_norm(h, stats, x, gn_weight.reshape(1, D),
                    gn_bias.reshape(1, D), gn_mean_scale.reshape(1, D))


# speedup vs baseline: 13.1399x; 1.0511x over previous
import functools

import jax
import jax.numpy as jnp
from jax import lax
from jax.experimental import pallas as pl
from jax.experimental.pallas import tpu as pltpu
from jax.experimental.pallas import tpu_sc as plsc

N = 10000
D = 128
E = 320000
NC = 2
NS = 16
NW = NC * NS
EPW = E // NW
CHUNK = 125
NCHUNK = EPW // CHUNK
BLK = 16
NBLK = NCHUNK // BLK
RPW = 624
NP = 10240


def _sc_body(x_hbm, src_hbm, dst_hbm, agg_out, cnt_out,
             srcA, dstA, srcB, dstB, rows, rows2, ones, zcnt,
             agg_sh, cnt_sh, siA, diA, siB, diB, sem, sem2):
    cid = lax.axis_index("c")
    sid = lax.axis_index("s")
    wid = cid * NS + sid

    def zr_body(r, carry):
        for c in range(D // 16):
            rows[r, pl.ds(c * 16, 16)] = jnp.zeros((16,), jnp.float32)
        return carry
    lax.fori_loop(0, CHUNK, zr_body, 0)

    def zc_body(i, carry):
        zcnt[pl.ds(i * 16, 16)] = jnp.zeros((16,), jnp.float32)
        return carry
    lax.fori_loop(0, 1024 // 16, zc_body, 0)

    for c in range(128 // 16):
        ones[pl.ds(c * 16, 16)] = jnp.ones((16,), jnp.float32)

    row0 = sid * RPW
    for k in range(RPW // CHUNK):
        pltpu.sync_copy(rows, agg_sh.at[pl.ds(row0 + k * CHUNK, CHUNK)])
    pltpu.sync_copy(rows.at[pl.ds(0, RPW % CHUNK)],
                    agg_sh.at[pl.ds(row0 + RPW - RPW % CHUNK, RPW % CHUNK)])

    @pl.when(sid == NS - 1)
    def _():
        pltpu.sync_copy(rows.at[pl.ds(0, 16)],
                        agg_sh.at[pl.ds(NS * RPW, 16)])

    @pl.when(sid == 0)
    def _():
        for k in range(NP // 1024):
            pltpu.sync_copy(zcnt, cnt_sh.at[pl.ds(k * 1024, 1024)])

    plsc.subcore_barrier()

    bufs = [(srcA, dstA, siA, diA), (srcB, dstB, siB, diB)]

    def stage(b, bi):
        sb, db, ss, ds_ = bufs[bi]
        return (pltpu.make_async_copy(src_hbm.at[wid, pl.ds(b * BLK, BLK)],
                                      sb, ss),
                pltpu.make_async_copy(dst_hbm.at[wid, pl.ds(b * BLK, BLK)],
                                      db, ds_))

    for c in stage(0, 0):
        c.start()
    for b in range(NBLK):
        bi = b % 2
        if b + 1 < NBLK:
            for c in stage(b + 1, (b + 1) % 2):
                c.start()
        for c in stage(b, bi):
            c.wait()
        sb, db = bufs[bi][0], bufs[bi][1]

        def gather(j, buf, s):
            return pltpu.make_async_copy(x_hbm.at[sb.at[j]], buf, s)

        gather(0, rows, sem).start()

        def pair_body(i, carry):
            j = i * 2
            gather(j + 1, rows2, sem2).start()
            gather(j, rows, sem).wait()
            pltpu.sync_copy(rows, agg_sh.at[db.at[j]], add=True)
            pltpu.sync_copy(ones.at[pl.ds(0, CHUNK)], cnt_sh.at[db.at[j]],
                            add=True)
            jn = jnp.minimum(j + 2, BLK - 1)
            gather(jn, rows, sem).start()
            gather(j + 1, rows2, sem2).wait()
            pltpu.sync_copy(rows2, agg_sh.at[db.at[j + 1]], add=True)
            pltpu.sync_copy(ones.at[pl.ds(0, CHUNK)],
                            cnt_sh.at[db.at[j + 1]], add=True)
            return carry
        lax.fori_loop(0, BLK // 2, pair_body, 0)
        gather(BLK - 1, rows, sem).wait()

    plsc.subcore_barrier()

    pltpu.sync_copy(agg_sh.at[pl.ds(row0, RPW)],
                    agg_out.at[cid, pl.ds(row0, RPW)])

    @pl.when(sid == NS - 1)
    def _():
        pltpu.sync_copy(agg_sh.at[pl.ds(NS * RPW, 16)],
                        agg_out.at[cid, pl.ds(NS * RPW, 16)])

    @pl.when(sid == 0)
    def _():
        pltpu.sync_copy(cnt_sh, cnt_out.at[pl.ds(cid * NP, NP)])


_sc_segment_sum = functools.partial(
    pl.kernel,
    out_type=(jax.ShapeDtypeStruct((NC, N, D), jnp.float32),
              jax.ShapeDtypeStruct((NC * NP,), jnp.float32)),
    mesh=plsc.VectorSubcoreMesh(core_axis_name="c", subcore_axis_name="s"),
    scratch_types=[
        pltpu.VMEM((BLK, CHUNK), jnp.int32),
        pltpu.VMEM((BLK, CHUNK), jnp.int32),
        pltpu.VMEM((BLK, CHUNK), jnp.int32),
        pltpu.VMEM((BLK, CHUNK), jnp.int32),
        pltpu.VMEM((CHUNK, D), jnp.float32),
        pltpu.VMEM((CHUNK, D), jnp.float32),
        pltpu.VMEM((128,), jnp.float32),
        pltpu.VMEM((1024,), jnp.float32),
        pltpu.VMEM_SHARED((N, D), jnp.float32),
        pltpu.VMEM_SHARED((NP,), jnp.float32),
        pltpu.SemaphoreType.DMA,
        pltpu.SemaphoreType.DMA,
        pltpu.SemaphoreType.DMA,
        pltpu.SemaphoreType.DMA,
        pltpu.SemaphoreType.DMA,
        pltpu.SemaphoreType.DMA,
    ],
)(_sc_body)


def _fused_body(agg_ref, cnt_ref, x_ref, wl_ref, wr_ref, bl_ref,
                w_ref, b_ref, ms_ref, o_ref):
    agg = agg_ref[0] + agg_ref[1]
    c = cnt_ref[0] + cnt_ref[1]
    mean = agg * (1.0 / jnp.maximum(c, 1.0))
    x = x_ref[...]
    dn = (((1,), (1,)), ((), ()))
    h = (lax.dot_general(mean, wl_ref[...], dn,
                         preferred_element_type=jnp.float32)
         + lax.dot_general(x, wr_ref[...], dn,
                           preferred_element_type=jnp.float32)
         + bl_ref[...])
    mu = jnp.sum(h, axis=0, keepdims=True) * (1.0 / N)
    m2 = jnp.sum(h * h, axis=0, keepdims=True) * (1.0 / N)
    mus = mu * ms_ref[...]
    var = m2 - 2.0 * mus * mu + mus * mus
    rstd = lax.rsqrt(var + 1e-5)
    hn = (h - mus) * rstd * w_ref[...] + b_ref[...]
    g = 0.5 * hn * (1.0 + lax.erf(hn * 0.7071067811865476))
    o_ref[...] = g + x


def _tc_fused(agg_parts, cnt_parts, x, W_l, W_r, b_l,
              gn_weight, gn_bias, gn_mean_scale):
    return pl.pallas_call(
        _fused_body,
        out_shape=jax.ShapeDtypeStruct((N, D), jnp.float32),
    )(agg_parts, cnt_parts, x, W_l, W_r, b_l,
      gn_weight, gn_bias, gn_mean_scale)


def kernel(x, edge_index, W_l, b_l, W_r, gn_weight, gn_bias, gn_mean_scale):
    src = edge_index[0].reshape(NW, NCHUNK, CHUNK)
    dst = edge_index[1].reshape(NW, NCHUNK, CHUNK)
    agg_parts, cnt_flat = _sc_segment_sum(x, src, dst)
    cnt_parts = cnt_flat.reshape(NC, NP)[:, :N].reshape(NC, N, 1)
    return _tc_fused(agg_parts, cnt_parts, x, W_l, W_r, b_l.reshape(1, D),
                     gn_weight.reshape(1, D), gn_bias.reshape(1, D),
                     gn_mean_scale.reshape(1, D))


# speedup vs baseline: 13.3169x; 1.0135x over previous
import functools

import jax
import jax.numpy as jnp
from jax import lax
from jax.experimental import pallas as pl
from jax.experimental.pallas import tpu as pltpu
from jax.experimental.pallas import tpu_sc as plsc

N = 10000
D = 128
E = 320000
NC = 2
NS = 16
NW = NC * NS
EPW = E // NW
CHUNK = 125
NCHUNK = EPW // CHUNK
BLK = 16
NBLK = NCHUNK // BLK
RPW = 624
NP = 10240


def _sc_body(x_hbm, src_hbm, dst_hbm, agg_out, cnt_out,
             srcA, dstA, srcB, dstB, rows, rows2, ones, zcnt,
             agg_sh, cnt_sh, siA, diA, siB, diB, sem, sem2, semc):
    cid = lax.axis_index("c")
    sid = lax.axis_index("s")
    wid = cid * NS + sid

    bufs = [(srcA, dstA, siA, diA), (srcB, dstB, siB, diB)]

    def stage(b, bi):
        sb, db, ss, ds_ = bufs[bi]
        return (pltpu.make_async_copy(src_hbm.at[wid, pl.ds(b * BLK, BLK)],
                                      sb, ss),
                pltpu.make_async_copy(dst_hbm.at[wid, pl.ds(b * BLK, BLK)],
                                      db, ds_))

    for c in stage(0, 0):
        c.start()

    def zr_body(r, carry):
        for c in range(D // 16):
            rows[r, pl.ds(c * 16, 16)] = jnp.zeros((16,), jnp.float32)
        return carry
    lax.fori_loop(0, CHUNK, zr_body, 0)

    def zc_body(i, carry):
        zcnt[pl.ds(i * 16, 16)] = jnp.zeros((16,), jnp.float32)
        return carry
    lax.fori_loop(0, 1024 // 16, zc_body, 0)

    for c in range(128 // 16):
        ones[pl.ds(c * 16, 16)] = jnp.ones((16,), jnp.float32)

    row0 = sid * RPW
    for k in range(RPW // CHUNK):
        pltpu.sync_copy(rows, agg_sh.at[pl.ds(row0 + k * CHUNK, CHUNK)])
    pltpu.sync_copy(rows.at[pl.ds(0, RPW % CHUNK)],
                    agg_sh.at[pl.ds(row0 + RPW - RPW % CHUNK, RPW % CHUNK)])

    @pl.when(sid == NS - 1)
    def _():
        pltpu.sync_copy(rows.at[pl.ds(0, 16)],
                        agg_sh.at[pl.ds(NS * RPW, 16)])

    @pl.when(sid == 0)
    def _():
        for k in range(NP // 1024):
            pltpu.sync_copy(zcnt, cnt_sh.at[pl.ds(k * 1024, 1024)])

    plsc.subcore_barrier()

    for b in range(NBLK):
        bi = b % 2
        if b + 1 < NBLK:
            for c in stage(b + 1, (b + 1) % 2):
                c.start()
        for c in stage(b, bi):
            c.wait()
        sb, db = bufs[bi][0], bufs[bi][1]

        def gather(j, buf, s):
            return pltpu.make_async_copy(x_hbm.at[sb.at[j]], buf, s)

        def count(j):
            return pltpu.async_copy(ones.at[pl.ds(0, CHUNK)],
                                    cnt_sh.at[db.at[j]], semc, add=True)

        gather(0, rows, sem).start()

        def count_body(j, carry):
            count(j)
            return carry
        lax.fori_loop(0, BLK, count_body, 0)

        def pair_body(i, carry):
            j = i * 2
            gather(j + 1, rows2, sem2).start()
            gather(j, rows, sem).wait()
            pltpu.sync_copy(rows, agg_sh.at[db.at[j]], add=True)
            jn = jnp.minimum(j + 2, BLK - 1)
            gather(jn, rows, sem).start()
            gather(j + 1, rows2, sem2).wait()
            pltpu.sync_copy(rows2, agg_sh.at[db.at[j + 1]], add=True)
            return carry
        lax.fori_loop(0, BLK // 2, pair_body, 0)
        gather(BLK - 1, rows, sem).wait()

        def count_drain(j, carry):
            pltpu.make_async_copy(ones.at[pl.ds(0, CHUNK)],
                                  cnt_sh.at[db.at[0]], semc).wait()
            return carry
        lax.fori_loop(0, BLK, count_drain, 0)

    plsc.subcore_barrier()

    pltpu.sync_copy(agg_sh.at[pl.ds(row0, RPW)],
                    agg_out.at[cid, pl.ds(row0, RPW)])

    @pl.when(sid == NS - 1)
    def _():
        pltpu.sync_copy(agg_sh.at[pl.ds(NS * RPW, 16)],
                        agg_out.at[cid, pl.ds(NS * RPW, 16)])

    @pl.when(sid == 0)
    def _():
        pltpu.sync_copy(cnt_sh, cnt_out.at[pl.ds(cid * NP, NP)])


_sc_segment_sum = functools.partial(
    pl.kernel,
    out_type=(jax.ShapeDtypeStruct((NC, N, D), jnp.float32),
              jax.ShapeDtypeStruct((NC * NP,), jnp.float32)),
    mesh=plsc.VectorSubcoreMesh(core_axis_name="c", subcore_axis_name="s"),
    scratch_types=[
        pltpu.VMEM((BLK, CHUNK), jnp.int32),
        pltpu.VMEM((BLK, CHUNK), jnp.int32),
        pltpu.VMEM((BLK, CHUNK), jnp.int32),
        pltpu.VMEM((BLK, CHUNK), jnp.int32),
        pltpu.VMEM((CHUNK, D), jnp.float32),
        pltpu.VMEM((CHUNK, D), jnp.float32),
        pltpu.VMEM((128,), jnp.float32),
        pltpu.VMEM((1024,), jnp.float32),
        pltpu.VMEM_SHARED((N, D), jnp.float32),
        pltpu.VMEM_SHARED((NP,), jnp.float32),
        pltpu.SemaphoreType.DMA,
        pltpu.SemaphoreType.DMA,
        pltpu.SemaphoreType.DMA,
        pltpu.SemaphoreType.DMA,
        pltpu.SemaphoreType.DMA,
        pltpu.SemaphoreType.DMA,
        pltpu.SemaphoreType.DMA,
    ],
)(_sc_body)


def _fused_body(agg_ref, cnt_ref, x_ref, wl_ref, wr_ref, bl_ref,
                w_ref, b_ref, ms_ref, o_ref):
    agg = agg_ref[0] + agg_ref[1]
    c = cnt_ref[0] + cnt_ref[1]
    mean = agg * (1.0 / jnp.maximum(c, 1.0))
    x = x_ref[...]
    dn = (((1,), (1,)), ((), ()))
    h = (lax.dot_general(mean, wl_ref[...], dn,
                         preferred_element_type=jnp.float32)
         + lax.dot_general(x, wr_ref[...], dn,
                           preferred_element_type=jnp.float32)
         + bl_ref[...])
    mu = jnp.sum(h, axis=0, keepdims=True) * (1.0 / N)
    m2 = jnp.sum(h * h, axis=0, keepdims=True) * (1.0 / N)
    mus = mu * ms_ref[...]
    var = m2 - 2.0 * mus * mu + mus * mus
    rstd = lax.rsqrt(var + 1e-5)
    hn = (h - mus) * rstd * w_ref[...] + b_ref[...]
    g = 0.5 * hn * (1.0 + lax.erf(hn * 0.7071067811865476))
    o_ref[...] = g + x


def _tc_fused(agg_parts, cnt_parts, x, W_l, W_r, b_l,
              gn_weight, gn_bias, gn_mean_scale):
    return pl.pallas_call(
        _fused_body,
        out_shape=jax.ShapeDtypeStruct((N, D), jnp.float32),
    )(agg_parts, cnt_parts, x, W_l, W_r, b_l,
      gn_weight, gn_bias, gn_mean_scale)


def kernel(x, edge_index, W_l, b_l, W_r, gn_weight, gn_bias, gn_mean_scale):
    src = edge_index[0].reshape(NW, NCHUNK, CHUNK)
    dst = edge_index[1].reshape(NW, NCHUNK, CHUNK)
    agg_parts, cnt_flat = _sc_segment_sum(x, src, dst)
    cnt_parts = cnt_flat.reshape(NC, NP)[:, :N].reshape(NC, N, 1)
    return _tc_fused(agg_parts, cnt_parts, x, W_l, W_r, b_l.reshape(1, D),
                     gn_weight.reshape(1, D), gn_bias.reshape(1, D),
                     gn_mean_scale.reshape(1, D))
